# baseline (speedup 1.0000x reference)
import functools

import jax
import jax.numpy as jnp
from jax import lax
from jax.experimental import pallas as pl
from jax.experimental.pallas import tpu as pltpu
from jax.experimental.pallas import tpu_sc as plsc

_L = 16
_G = 128


def _sc_info():
    try:
        info = plsc.get_sparse_core_info()
        return info.num_cores, info.num_subcores
    except Exception:
        return 2, 16


def _compact_body(nw, rpt, nb, blk, cpad, ei_hbm, srcc_hbm, ldst_hbm,
                  cnts_hbm, dbuf, sbuf, cs, cl, cnt_v):
    nc, _ = _sc_info()
    wid = lax.axis_index("s") * nc + lax.axis_index("c")
    lo = wid * rpt
    hi = lo + rpt
    nch = blk // _L

    def zero_chunk(i, _):
        cs[pl.ds(i * _L, _L)] = jnp.zeros((_L,), jnp.int32)
        return 0

    lax.fori_loop(0, nch, zero_chunk, 0)

    def block(b, _):
        pltpu.sync_copy(ei_hbm.at[0, pl.ds(b * blk, blk)], sbuf)
        pltpu.sync_copy(ei_hbm.at[1, pl.ds(b * blk, blk)], dbuf)

        def chunk2(i, off):
            d16a = dbuf[pl.ds((2 * i) * _L, _L)]
            s16a = sbuf[pl.ds((2 * i) * _L, _L)]
            d16b = dbuf[pl.ds((2 * i + 1) * _L, _L)]
            s16b = sbuf[pl.ds((2 * i + 1) * _L, _L)]
            ma = (d16a >= lo) & (d16a < hi)
            mb = (d16b >= lo) & (d16b < hi)
            mia = jnp.where(ma, 1, 0).astype(jnp.int32)
            mib = jnp.where(mb, 1, 0).astype(jnp.int32)
            incla = plsc.cumsum(mia)
            inclb = plsc.cumsum(mib)
            cnta = incla[_L - 1]
            idxa = jnp.where(ma, off + incla - mia, blk)
            idxb = jnp.where(mb, off + cnta + inclb - mib, blk)
            plsc.store_scatter(cs, [idxa], s16a)
            plsc.store_scatter(cl, [idxa], d16a - lo)
            plsc.store_scatter(cs, [idxb], s16b)
            plsc.store_scatter(cl, [idxb], d16b - lo)
            return off + cnta + inclb[_L - 1]

        cnt = lax.fori_loop(0, nch // 2, chunk2, 0)
        lane = lax.iota(jnp.int32, _L)
        cidx = jnp.where(lane == 0, b, cpad - 1)
        plsc.store_scatter(cnt_v, [cidx], jnp.full((_L,), cnt, jnp.int32))
        pltpu.sync_copy(cs.at[pl.ds(0, blk)], srcc_hbm.at[wid, b])
        pltpu.sync_copy(cl.at[pl.ds(0, blk)], ldst_hbm.at[wid, b])
        return 0

    lax.fori_loop(0, nb, block, 0)
    pltpu.sync_copy(cnt_v, cnts_hbm.at[wid])


def _segmax_body(nw, rpt, nb, blk, h_hbm, srcc_hbm, ldst_hbm, cnts_hbm,
                 out_hbm, cnt_v, sbuf, lbuf, rows, rows2, acc, sem, sem2):
    nc, _ = _sc_info()
    wid = lax.axis_index("s") * nc + lax.axis_index("c")
    lo = wid * rpt
    nvec = rpt * 128 // _L
    neg = jnp.full((_L,), -jnp.inf, jnp.float32)

    pltpu.sync_copy(cnts_hbm.at[wid], cnt_v)

    def init_chunk(i, _):
        acc[pl.ds(i * _L, _L)] = neg
        return 0

    lax.fori_loop(0, nvec, init_chunk, 0)

    def block(b, _):
        c = cnt_v[pl.ds(b, _L)][0]

        @pl.when(c > 0)
        def _():
            pltpu.sync_copy(srcc_hbm.at[wid, b], sbuf)
            pltpu.sync_copy(ldst_hbm.at[wid, b], lbuf.at[pl.ds(0, blk)])
            ngather = (c + _G - 1) // _G

            def start(k, buf, s):
                pltpu.async_copy(h_hbm.at[sbuf.at[pl.ds(k * _G, _G)]], buf, s)

            def process(k, buf):
                ne = jnp.minimum(_G, c - k * _G)

                def upd(e, d):
                    base = d * 128
                    avs = [acc[pl.ds(base + j * _L, _L)] for j in range(8)]
                    rvs = [buf[e, pl.ds(j * _L, _L)] for j in range(8)]
                    for j in range(8):
                        acc[pl.ds(base + j * _L, _L)] = jnp.maximum(
                            avs[j], rvs[j])

                def group16(g, _):
                    dvec = lbuf[pl.ds(k * _G + g * _L, _L)]
                    for e2 in range(_L):
                        upd(g * _L + e2, dvec[e2])
                    return 0

                def edge(e, _):
                    upd(e, lbuf[pl.ds(k * _G + e, _L)][0])
                    return 0

                nfull = ne // _L
                lax.fori_loop(0, nfull, group16, 0)
                lax.fori_loop(nfull * _L, ne, edge, 0)

            def wait(buf, s):
                pltpu.make_async_copy(h_hbm.at[sbuf.at[pl.ds(0, _G)]],
                                      buf, s).wait()

            start(0, rows, sem)

            def pair(p, _):
                k0, k1 = 2 * p, 2 * p + 1

                @pl.when(k1 < ngather)
                def _():
                    start(k1, rows2, sem2)

                wait(rows, sem)
                process(k0, rows)

                @pl.when(k1 < ngather)
                def _():
                    @pl.when(k1 + 1 < ngather)
                    def _():
                        start(k1 + 1, rows, sem)

                    wait(rows2, sem2)
                    process(k1, rows2)

                return 0

            lax.fori_loop(0, (ngather + 1) // 2, pair, 0)

        return 0

    lax.fori_loop(0, nb, block, 0)

    def fix_chunk(i, _):
        a = acc[pl.ds(i * _L, _L)]
        acc[pl.ds(i * _L, _L)] = jnp.where(a == neg, 0.0, a)
        return 0

    lax.fori_loop(0, nvec, fix_chunk, 0)
    pltpu.sync_copy(acc, out_hbm.at[pl.ds(lo * 128, rpt * 128)])


def _sage_lin_tc(agg, h, wlT, bl, wrT):
    n = agg.shape[0]

    def body(a_ref, h_ref, wl_ref, bl_ref, wr_ref, o_ref):
        o_ref[...] = (
            jnp.dot(a_ref[...], wl_ref[...], preferred_element_type=jnp.float32)
            + jnp.dot(h_ref[...], wr_ref[...], preferred_element_type=jnp.float32)
            + bl_ref[...])

    return pl.pallas_call(
        body, out_shape=jax.ShapeDtypeStruct((n, wlT.shape[1]), jnp.float32),
    )(agg, h, wlT, bl.reshape(1, -1), wrT)


def _final_tc(agg, h, wlT, bl, wrT, w1T, b1, w2T, b2, w3T, b3):
    n = agg.shape[0]

    def body(a_ref, h_ref, wl_ref, bl_ref, wr_ref, w1_ref, b1_ref, w2_ref,
             b2_ref, w3_ref, b3_ref, o_ref):
        h3 = (jnp.dot(a_ref[...], wl_ref[...], preferred_element_type=jnp.float32)
              + jnp.dot(h_ref[...], wr_ref[...], preferred_element_type=jnp.float32)
              + bl_ref[...])
        t = jnp.maximum(
            jnp.dot(h3, w1_ref[...], preferred_element_type=jnp.float32)
            + b1_ref[...], 0.0)
        t = jnp.maximum(
            jnp.dot(t, w2_ref[...], preferred_element_type=jnp.float32)
            + b2_ref[...], 0.0)
        t = (jnp.dot(t, w3_ref[...], preferred_element_type=jnp.float32)
             + b3_ref[...])
        o_ref[...] = 1.0 / (1.0 + jnp.exp(-t))

    return pl.pallas_call(
        body, out_shape=jax.ShapeDtypeStruct((n, w3T.shape[1]), jnp.float32),
    )(agg, h, wlT, bl.reshape(1, -1), wrT, w1T, b1.reshape(1, -1), w2T,
      b2.reshape(1, -1), w3T, b3.reshape(1, -1))


def kernel(x, edge_index, batch, W1l, b1l, W1r, W2l, b2l, W2r, W3l, b3l, W3r,
           l1W, l1b, l2W, l2b, l3W, l3b):
    n, d = x.shape
    e = edge_index.shape[1]
    nc, ns = _sc_info()
    nw = nc * ns
    rpt = -(-n // nw)
    rpt = -(-rpt // 8) * 8
    n2 = nw * rpt

    blk = 12800
    while e % blk:
        blk //= 2
    nb = e // blk
    cpad = -(-nb // _L) * _L + _L

    mesh = plsc.VectorSubcoreMesh(core_axis_name="c", subcore_axis_name="s")

    compact = pl.kernel(
        functools.partial(_compact_body, nw, rpt, nb, blk, cpad),
        out_type=(
            jax.ShapeDtypeStruct((nw, nb, blk), jnp.int32),
            jax.ShapeDtypeStruct((nw, nb, blk), jnp.int32),
            jax.ShapeDtypeStruct((nw, cpad), jnp.int32),
        ),
        mesh=mesh,
        scratch_types=[
            pltpu.VMEM((blk,), jnp.int32),
            pltpu.VMEM((blk,), jnp.int32),
            pltpu.VMEM((blk + _L,), jnp.int32),
            pltpu.VMEM((blk + _L,), jnp.int32),
            pltpu.VMEM((cpad,), jnp.int32),
        ],
        compiler_params=pltpu.CompilerParams(needs_layout_passes=False),
    )

    segmax = pl.kernel(
        functools.partial(_segmax_body, nw, rpt, nb, blk),
        out_type=jax.ShapeDtypeStruct((n2 * 128,), jnp.float32),
        mesh=mesh,
        scratch_types=[
            pltpu.VMEM((cpad,), jnp.int32),
            pltpu.VMEM((blk,), jnp.int32),
            pltpu.VMEM((blk + _L,), jnp.int32),
            pltpu.VMEM((_G, 128), jnp.float32),
            pltpu.VMEM((_G, 128), jnp.float32),
            pltpu.VMEM((rpt * 128,), jnp.float32),
            pltpu.SemaphoreType.DMA,
            pltpu.SemaphoreType.DMA,
        ],
        compiler_params=pltpu.CompilerParams(needs_layout_passes=False),
    )

    src_c, ldst_c, cnts = compact(edge_index)

    def agg_of(hcur):
        flat = segmax(hcur, src_c, ldst_c, cnts)
        return flat.reshape(n2, 128)[:n]

    w3p = jnp.zeros((8, l3W.shape[1]), jnp.float32).at[:l3W.shape[0]].set(l3W)
    b3p = jnp.zeros((8,), jnp.float32).at[:l3b.shape[0]].set(l3b)

    h1 = _sage_lin_tc(agg_of(x), x, W1l.T, b1l, W1r.T)
    h2 = _sage_lin_tc(agg_of(h1), h1, W2l.T, b2l, W2r.T)
    out = _final_tc(agg_of(h2), h2, W3l.T, b3l, W3r.T, l1W.T, l1b, l2W.T,
                    l2b, w3p.T, b3p)
    return out[:, :l3W.shape[0]]


# speedup vs baseline: 1.8223x; 1.8223x over previous
import functools

import jax
import jax.numpy as jnp
from jax import lax
from jax.experimental import pallas as pl
from jax.experimental.pallas import tpu as pltpu
from jax.experimental.pallas import tpu_sc as plsc

_L = 16


def _sc_info():
    try:
        info = plsc.get_sparse_core_info()
        return info.num_cores, info.num_subcores
    except Exception:
        return 2, 16


def _segmax_t_body(nw, w, fpt, e, c, ei_hbm, ht_hbm, out_hbm,
                   hts, acc, tmp, sa, da, sb, db, sema, semb):
    nc, _ = _sc_info()
    wid = lax.axis_index("s") * nc + lax.axis_index("c")
    slab = fpt * w
    base = wid * slab
    neg = jnp.full((_L,), -jnp.inf, jnp.float32)
    lane = lax.iota(jnp.int32, _L)
    nch = e // c

    pltpu.sync_copy(ht_hbm.at[pl.ds(base, slab)], hts)

    def init_chunk(i, _):
        acc[pl.ds(i * _L, _L)] = neg
        return 0

    lax.fori_loop(0, slab // _L, init_chunk, 0)

    def start(ci, s_buf, d_buf, sem):
        pltpu.async_copy(ei_hbm.at[0, pl.ds(ci * c, c)], s_buf, sem)
        pltpu.async_copy(ei_hbm.at[1, pl.ds(ci * c, c)], d_buf, sem)

    def wait(s_buf, d_buf, sem):
        pltpu.make_async_copy(ei_hbm.at[0, pl.ds(0, c)], s_buf, sem).wait()
        pltpu.make_async_copy(ei_hbm.at[0, pl.ds(0, c)], d_buf, sem).wait()

    def process(s_buf, d_buf):
        def group(g, _):
            s16 = s_buf[pl.ds(g * _L, _L)]
            d16 = d_buf[pl.ds(g * _L, _L)]
            hvs = [plsc.load_gather(hts, [s16 + f * w]) for f in range(fpt)]

            def cond(carry):
                rem, r = carry
                pc = plsc.all_reduce_population_count(rem)
                pcs = pc if pc.ndim == 0 else pc[0]
                return (pcs > 0) & (r < _L)

            def wbody(carry):
                rem, r = carry
                idxw = jnp.where(rem, d16, w)
                plsc.store_scatter(tmp, [idxw], lane)
                back = plsc.load_gather(tmp, [idxw])
                winner = rem & (back == lane)
                for f in range(fpt):
                    idxu = jnp.where(winner, d16 + f * w, slab)
                    av = plsc.load_gather(acc, [idxu])
                    plsc.store_scatter(acc, [idxu],
                                       jnp.maximum(av, hvs[f]))
                return rem & (~winner), r + 1

            lax.while_loop(cond, wbody, (d16 >= 0, 0))
            return 0

        lax.fori_loop(0, c // _L, group, 0)

    start(0, sa, da, sema)

    def pairbody(p, _):
        start(2 * p + 1, sb, db, semb)
        wait(sa, da, sema)
        process(sa, da)
        start(2 * p + 2, sa, da, sema)
        wait(sb, db, semb)
        process(sb, db)
        return 0

    lax.fori_loop(0, (nch - 1) // 2, pairbody, 0)
    wait(sa, da, sema)
    process(sa, da)

    def fix_chunk(i, _):
        a = acc[pl.ds(i * _L, _L)]
        acc[pl.ds(i * _L, _L)] = jnp.where(a == neg, 0.0, a)
        return 0

    lax.fori_loop(0, slab // _L, fix_chunk, 0)
    pltpu.sync_copy(acc.at[pl.ds(0, slab)], out_hbm.at[pl.ds(base, slab)])


def _transpose_tc(x):
    def body(x_ref, o_ref):
        o_ref[...] = x_ref[...].T

    return pl.pallas_call(
        body,
        out_shape=jax.ShapeDtypeStruct((x.shape[1], x.shape[0]), jnp.float32),
    )(x)


def _layer_tc(aggT, hT, wl, bl, wr):
    def body(a_ref, h_ref, wl_ref, bl_ref, wr_ref, o_ref):
        o_ref[...] = (
            jnp.dot(wl_ref[...], a_ref[...], preferred_element_type=jnp.float32)
            + jnp.dot(wr_ref[...], h_ref[...], preferred_element_type=jnp.float32)
            + bl_ref[...])

    return pl.pallas_call(
        body, out_shape=jax.ShapeDtypeStruct(aggT.shape, jnp.float32),
    )(aggT, hT, wl, bl.reshape(-1, 1), wr)


def _head_tc(aggT, hT, wl, bl, wr, w1, b1, w2, b2, w3p, b3p):
    def body(a_ref, h_ref, wl_ref, bl_ref, wr_ref, w1_ref, b1_ref, w2_ref,
             b2_ref, w3_ref, b3_ref, o_ref):
        h3 = (jnp.dot(wl_ref[...], a_ref[...], preferred_element_type=jnp.float32)
              + jnp.dot(wr_ref[...], h_ref[...], preferred_element_type=jnp.float32)
              + bl_ref[...])
        t = jnp.maximum(
            jnp.dot(w1_ref[...], h3, preferred_element_type=jnp.float32)
            + b1_ref[...], 0.0)
        t = jnp.maximum(
            jnp.dot(w2_ref[...], t, preferred_element_type=jnp.float32)
            + b2_ref[...], 0.0)
        t = (jnp.dot(w3_ref[...], t, preferred_element_type=jnp.float32)
             + b3_ref[...])
        o_ref[...] = 1.0 / (1.0 + jnp.exp(-t))

    return pl.pallas_call(
        body,
        out_shape=jax.ShapeDtypeStruct((w3p.shape[0], aggT.shape[1]),
                                       jnp.float32),
    )(aggT, hT, wl, bl.reshape(-1, 1), wr, w1, b1.reshape(-1, 1), w2,
      b2.reshape(-1, 1), w3p, b3p.reshape(-1, 1))


def kernel(x, edge_index, batch, W1l, b1l, W1r, W2l, b2l, W2r, W3l, b3l, W3r,
           l1W, l1b, l2W, l2b, l3W, l3b):
    n, d = x.shape
    e = edge_index.shape[1]
    nc, ns = _sc_info()
    nw = nc * ns
    fpt = d // nw

    c = 2560
    while e % c:
        c //= 2

    mesh = plsc.VectorSubcoreMesh(core_axis_name="c", subcore_axis_name="s")

    segmax = pl.kernel(
        functools.partial(_segmax_t_body, nw, n, fpt, e, c),
        out_type=jax.ShapeDtypeStruct((d * n,), jnp.float32),
        mesh=mesh,
        scratch_types=[
            pltpu.VMEM((fpt * n,), jnp.float32),
            pltpu.VMEM((fpt * n + _L,), jnp.float32),
            pltpu.VMEM((n + _L,), jnp.int32),
            pltpu.VMEM((c,), jnp.int32),
            pltpu.VMEM((c,), jnp.int32),
            pltpu.VMEM((c,), jnp.int32),
            pltpu.VMEM((c,), jnp.int32),
            pltpu.SemaphoreType.DMA,
            pltpu.SemaphoreType.DMA,
        ],
        compiler_params=pltpu.CompilerParams(needs_layout_passes=False),
    )

    def agg_of(hT):
        return segmax(edge_index, hT.reshape(-1)).reshape(d, n)

    w3p = jnp.zeros((8, l3W.shape[1]), jnp.float32).at[:l3W.shape[0]].set(l3W)
    b3p = jnp.zeros((8,), jnp.float32).at[:l3b.shape[0]].set(l3b)

    xT = _transpose_tc(x)
    h1T = _layer_tc(agg_of(xT), xT, W1l, b1l, W1r)
    h2T = _layer_tc(agg_of(h1T), h1T, W2l, b2l, W2r)
    outT = _head_tc(agg_of(h2T), h2T, W3l, b3l, W3r, l1W, l1b, l2W, l2b,
                    w3p, b3p)
    return outT[:l3W.shape[0]].T


# speedup vs baseline: 2.5715x; 1.4111x over previous
import functools

import jax
import jax.numpy as jnp
from jax import lax
from jax.experimental import pallas as pl
from jax.experimental.pallas import tpu as pltpu
from jax.experimental.pallas import tpu_sc as plsc

_L = 16


def _sc_info():
    try:
        info = plsc.get_sparse_core_info()
        return info.num_cores, info.num_subcores
    except Exception:
        return 2, 16


def _segmax_t_body(nw, w, fpt, e, c, ei_hbm, ht_hbm, out_hbm,
                   hts, acc, tmp, sa, da, sb, db, sema, semb):
    nc, _ = _sc_info()
    wid = lax.axis_index("s") * nc + lax.axis_index("c")
    slab = fpt * w
    base = wid * slab
    neg = jnp.full((_L,), -jnp.inf, jnp.float32)
    lane = lax.iota(jnp.int32, _L)
    nch = e // c

    pltpu.sync_copy(ht_hbm.at[pl.ds(base, slab)], hts)

    def init_chunk(i, _):
        acc[pl.ds(i * _L, _L)] = neg
        return 0

    lax.fori_loop(0, slab // _L, init_chunk, 0)

    def start(ci, s_buf, d_buf, sem):
        pltpu.async_copy(ei_hbm.at[0, pl.ds(ci * c, c)], s_buf, sem)
        pltpu.async_copy(ei_hbm.at[1, pl.ds(ci * c, c)], d_buf, sem)

    def wait(s_buf, d_buf, sem):
        pltpu.make_async_copy(ei_hbm.at[0, pl.ds(0, c)], s_buf, sem).wait()
        pltpu.make_async_copy(ei_hbm.at[0, pl.ds(0, c)], d_buf, sem).wait()

    def process(s_buf, d_buf):
        def group(g, _):
            s16 = s_buf[pl.ds(g * _L, _L)]
            d16 = d_buf[pl.ds(g * _L, _L)]
            hvs = [plsc.load_gather(hts, [s16 + f * w]) for f in range(fpt)]

            def round_(rem):
                idxw = jnp.where(rem, d16, w)
                plsc.store_scatter(tmp, [idxw], lane)
                back = plsc.load_gather(tmp, [idxw])
                winner = rem & (back == lane)
                for f in range(fpt):
                    idxu = jnp.where(winner, d16 + f * w, slab)
                    av = plsc.load_gather(acc, [idxu])
                    plsc.store_scatter(acc, [idxu],
                                       jnp.maximum(av, hvs[f]))
                return rem & (~winner)

            rem0 = round_(d16 >= 0)

            @pl.when(jnp.any(rem0))
            def _():
                def cond(carry):
                    rem, r = carry
                    return jnp.any(rem) & (r < _L)

                def wbody(carry):
                    rem, r = carry
                    return round_(rem), r + 1

                lax.while_loop(cond, wbody, (rem0, 0))

            return 0

        lax.fori_loop(0, c // _L, group, 0)

    start(0, sa, da, sema)

    def pairbody(p, _):
        start(2 * p + 1, sb, db, semb)
        wait(sa, da, sema)
        process(sa, da)
        start(2 * p + 2, sa, da, sema)
        wait(sb, db, semb)
        process(sb, db)
        return 0

    lax.fori_loop(0, (nch - 1) // 2, pairbody, 0)
    wait(sa, da, sema)
    process(sa, da)

    def fix_chunk(i, _):
        a = acc[pl.ds(i * _L, _L)]
        acc[pl.ds(i * _L, _L)] = jnp.where(a == neg, 0.0, a)
        return 0

    lax.fori_loop(0, slab // _L, fix_chunk, 0)
    pltpu.sync_copy(acc.at[pl.ds(0, slab)], out_hbm.at[pl.ds(base, slab)])


def _transpose_tc(x):
    def body(x_ref, o_ref):
        o_ref[...] = x_ref[...].T

    return pl.pallas_call(
        body,
        out_shape=jax.ShapeDtypeStruct((x.shape[1], x.shape[0]), jnp.float32),
    )(x)


def _layer_tc(aggT, hT, wl, bl, wr):
    def body(a_ref, h_ref, wl_ref, bl_ref, wr_ref, o_ref):
        o_ref[...] = (
            jnp.dot(wl_ref[...], a_ref[...], preferred_element_type=jnp.float32)
            + jnp.dot(wr_ref[...], h_ref[...], preferred_element_type=jnp.float32)
            + bl_ref[...])

    return pl.pallas_call(
        body, out_shape=jax.ShapeDtypeStruct(aggT.shape, jnp.float32),
    )(aggT, hT, wl, bl.reshape(-1, 1), wr)


def _head_tc(aggT, hT, wl, bl, wr, w1, b1, w2, b2, w3p, b3p):
    def body(a_ref, h_ref, wl_ref, bl_ref, wr_ref, w1_ref, b1_ref, w2_ref,
             b2_ref, w3_ref, b3_ref, o_ref):
        h3 = (jnp.dot(wl_ref[...], a_ref[...], preferred_element_type=jnp.float32)
              + jnp.dot(wr_ref[...], h_ref[...], preferred_element_type=jnp.float32)
              + bl_ref[...])
        t = jnp.maximum(
            jnp.dot(w1_ref[...], h3, preferred_element_type=jnp.float32)
            + b1_ref[...], 0.0)
        t = jnp.maximum(
            jnp.dot(w2_ref[...], t, preferred_element_type=jnp.float32)
            + b2_ref[...], 0.0)
        t = (jnp.dot(w3_ref[...], t, preferred_element_type=jnp.float32)
             + b3_ref[...])
        o_ref[...] = 1.0 / (1.0 + jnp.exp(-t))

    return pl.pallas_call(
        body,
        out_shape=jax.ShapeDtypeStruct((w3p.shape[0], aggT.shape[1]),
                                       jnp.float32),
    )(aggT, hT, wl, bl.reshape(-1, 1), wr, w1, b1.reshape(-1, 1), w2,
      b2.reshape(-1, 1), w3p, b3p.reshape(-1, 1))


def kernel(x, edge_index, batch, W1l, b1l, W1r, W2l, b2l, W2r, W3l, b3l, W3r,
           l1W, l1b, l2W, l2b, l3W, l3b):
    n, d = x.shape
    e = edge_index.shape[1]
    nc, ns = _sc_info()
    nw = nc * ns
    fpt = d // nw

    c = 2560
    while e % c:
        c //= 2

    mesh = plsc.VectorSubcoreMesh(core_axis_name="c", subcore_axis_name="s")

    segmax = pl.kernel(
        functools.partial(_segmax_t_body, nw, n, fpt, e, c),
        out_type=jax.ShapeDtypeStruct((d * n,), jnp.float32),
        mesh=mesh,
        scratch_types=[
            pltpu.VMEM((fpt * n,), jnp.float32),
            pltpu.VMEM((fpt * n + _L,), jnp.float32),
            pltpu.VMEM((n + _L,), jnp.int32),
            pltpu.VMEM((c,), jnp.int32),
            pltpu.VMEM((c,), jnp.int32),
            pltpu.VMEM((c,), jnp.int32),
            pltpu.VMEM((c,), jnp.int32),
            pltpu.SemaphoreType.DMA,
            pltpu.SemaphoreType.DMA,
        ],
        compiler_params=pltpu.CompilerParams(needs_layout_passes=False),
    )

    def agg_of(hT):
        return segmax(edge_index, hT.reshape(-1)).reshape(d, n)

    w3p = jnp.zeros((8, l3W.shape[1]), jnp.float32).at[:l3W.shape[0]].set(l3W)
    b3p = jnp.zeros((8,), jnp.float32).at[:l3b.shape[0]].set(l3b)

    xT = _transpose_tc(x)
    h1T = _layer_tc(agg_of(xT), xT, W1l, b1l, W1r)
    h2T = _layer_tc(agg_of(h1T), h1T, W2l, b2l, W2r)
    outT = _head_tc(agg_of(h2T), h2T, W3l, b3l, W3r, l1W, l1b, l2W, l2b,
                    w3p, b3p)
    return outT[:l3W.shape[0]].T


# speedup vs baseline: 3.4138x; 1.3275x over previous
import functools

import jax
import jax.numpy as jnp
from jax import lax
from jax.experimental import pallas as pl
from jax.experimental.pallas import tpu as pltpu
from jax.experimental.pallas import tpu_sc as plsc

_L = 16


def _sc_info():
    try:
        info = plsc.get_sparse_core_info()
        return info.num_cores, info.num_subcores
    except Exception:
        return 2, 16


def _segmax_t_body(nw, w, fpt, e, c, ei_hbm, ht_hbm, out_hbm,
                   hts, acc, tmp, sa, da, sb, db, sema, semb):
    nc, _ = _sc_info()
    wid = lax.axis_index("s") * nc + lax.axis_index("c")
    slab = fpt * w
    base = wid * slab
    neg = jnp.full((_L,), -jnp.inf, jnp.float32)
    lane = lax.iota(jnp.int32, _L)
    nch = e // c

    pltpu.sync_copy(ht_hbm.at[pl.ds(base, slab)], hts)

    def init_chunk(i, _):
        acc[pl.ds(i * _L, _L)] = neg
        return 0

    lax.fori_loop(0, slab // _L, init_chunk, 0)

    def start(ci, s_buf, d_buf, sem):
        pltpu.async_copy(ei_hbm.at[0, pl.ds(ci * c, c)], s_buf, sem)
        pltpu.async_copy(ei_hbm.at[1, pl.ds(ci * c, c)], d_buf, sem)

    def wait(s_buf, d_buf, sem):
        pltpu.make_async_copy(ei_hbm.at[0, pl.ds(0, c)], s_buf, sem).wait()
        pltpu.make_async_copy(ei_hbm.at[0, pl.ds(0, c)], d_buf, sem).wait()

    def process(s_buf, d_buf):
        unroll = 4

        def round_(s16, d16, hvs, rem):
            idxw = jnp.where(rem, d16, w)
            plsc.store_scatter(tmp, [idxw], lane)
            back = plsc.load_gather(tmp, [idxw])
            winner = rem & (back == lane)
            idxus = [jnp.where(winner, d16 + f * w, slab) for f in range(fpt)]
            avs = [plsc.load_gather(acc, [idxus[f]]) for f in range(fpt)]
            mxs = [jnp.maximum(avs[f], hvs[f]) for f in range(fpt)]
            for f in range(fpt):
                plsc.store_scatter(acc, [idxus[f]], mxs[f])
            return rem & (~winner)

        def group4(q, _):
            rems = []
            for u in range(unroll):
                gb = (q * unroll + u) * _L
                s16 = s_buf[pl.ds(gb, _L)]
                d16 = d_buf[pl.ds(gb, _L)]
                hvs = [plsc.load_gather(hts, [s16 + f * w])
                       for f in range(fpt)]
                rems.append(round_(s16, d16, hvs, d16 >= 0))

            anyrem = rems[0]
            for u in range(1, unroll):
                anyrem = anyrem | rems[u]

            @pl.when(jnp.any(anyrem))
            def _():
                for u in range(unroll):
                    gb = (q * unroll + u) * _L
                    s16 = s_buf[pl.ds(gb, _L)]
                    d16 = d_buf[pl.ds(gb, _L)]
                    hvs = [plsc.load_gather(hts, [s16 + f * w])
                           for f in range(fpt)]

                    def cond(carry):
                        rem, r = carry
                        return jnp.any(rem) & (r < _L)

                    def wbody(carry):
                        rem, r = carry
                        return round_(s16, d16, hvs, rem), r + 1

                    lax.while_loop(cond, wbody, (d16 >= 0, 0))

            return 0

        lax.fori_loop(0, c // (_L * unroll), group4, 0)

    start(0, sa, da, sema)

    def pairbody(p, _):
        start(2 * p + 1, sb, db, semb)
        wait(sa, da, sema)
        process(sa, da)
        start(2 * p + 2, sa, da, sema)
        wait(sb, db, semb)
        process(sb, db)
        return 0

    lax.fori_loop(0, (nch - 1) // 2, pairbody, 0)
    wait(sa, da, sema)
    process(sa, da)

    def fix_chunk(i, _):
        a = acc[pl.ds(i * _L, _L)]
        acc[pl.ds(i * _L, _L)] = jnp.where(a == neg, 0.0, a)
        return 0

    lax.fori_loop(0, slab // _L, fix_chunk, 0)
    pltpu.sync_copy(acc.at[pl.ds(0, slab)], out_hbm.at[pl.ds(base, slab)])


def _transpose_tc(x):
    def body(x_ref, o_ref):
        o_ref[...] = x_ref[...].T

    return pl.pallas_call(
        body,
        out_shape=jax.ShapeDtypeStruct((x.shape[1], x.shape[0]), jnp.float32),
    )(x)


def _layer_tc(aggT, hT, wl, bl, wr):
    def body(a_ref, h_ref, wl_ref, bl_ref, wr_ref, o_ref):
        o_ref[...] = (
            jnp.dot(wl_ref[...], a_ref[...], preferred_element_type=jnp.float32)
            + jnp.dot(wr_ref[...], h_ref[...], preferred_element_type=jnp.float32)
            + bl_ref[...])

    return pl.pallas_call(
        body, out_shape=jax.ShapeDtypeStruct(aggT.shape, jnp.float32),
    )(aggT, hT, wl, bl.reshape(-1, 1), wr)


def _head_tc(aggT, hT, wl, bl, wr, w1, b1, w2, b2, w3p, b3p):
    def body(a_ref, h_ref, wl_ref, bl_ref, wr_ref, w1_ref, b1_ref, w2_ref,
             b2_ref, w3_ref, b3_ref, o_ref):
        h3 = (jnp.dot(wl_ref[...], a_ref[...], preferred_element_type=jnp.float32)
              + jnp.dot(wr_ref[...], h_ref[...], preferred_element_type=jnp.float32)
              + bl_ref[...])
        t = jnp.maximum(
            jnp.dot(w1_ref[...], h3, preferred_element_type=jnp.float32)
            + b1_ref[...], 0.0)
        t = jnp.maximum(
            jnp.dot(w2_ref[...], t, preferred_element_type=jnp.float32)
            + b2_ref[...], 0.0)
        t = (jnp.dot(w3_ref[...], t, preferred_element_type=jnp.float32)
             + b3_ref[...])
        o_ref[...] = 1.0 / (1.0 + jnp.exp(-t))

    return pl.pallas_call(
        body,
        out_shape=jax.ShapeDtypeStruct((w3p.shape[0], aggT.shape[1]),
                                       jnp.float32),
    )(aggT, hT, wl, bl.reshape(-1, 1), wr, w1, b1.reshape(-1, 1), w2,
      b2.reshape(-1, 1), w3p, b3p.reshape(-1, 1))


def kernel(x, edge_index, batch, W1l, b1l, W1r, W2l, b2l, W2r, W3l, b3l, W3r,
           l1W, l1b, l2W, l2b, l3W, l3b):
    n, d = x.shape
    e = edge_index.shape[1]
    nc, ns = _sc_info()
    nw = nc * ns
    fpt = d // nw

    c = 2560
    while e % c:
        c //= 2

    mesh = plsc.VectorSubcoreMesh(core_axis_name="c", subcore_axis_name="s")

    segmax = pl.kernel(
        functools.partial(_segmax_t_body, nw, n, fpt, e, c),
        out_type=jax.ShapeDtypeStruct((d * n,), jnp.float32),
        mesh=mesh,
        scratch_types=[
            pltpu.VMEM((fpt * n,), jnp.float32),
            pltpu.VMEM((fpt * n + _L,), jnp.float32),
            pltpu.VMEM((n + _L,), jnp.int32),
            pltpu.VMEM((c,), jnp.int32),
            pltpu.VMEM((c,), jnp.int32),
            pltpu.VMEM((c,), jnp.int32),
            pltpu.VMEM((c,), jnp.int32),
            pltpu.SemaphoreType.DMA,
            pltpu.SemaphoreType.DMA,
        ],
        compiler_params=pltpu.CompilerParams(needs_layout_passes=False),
    )

    def agg_of(hT):
        return segmax(edge_index, hT.reshape(-1)).reshape(d, n)

    w3p = jnp.zeros((8, l3W.shape[1]), jnp.float32).at[:l3W.shape[0]].set(l3W)
    b3p = jnp.zeros((8,), jnp.float32).at[:l3b.shape[0]].set(l3b)

    xT = _transpose_tc(x)
    h1T = _layer_tc(agg_of(xT), xT, W1l, b1l, W1r)
    h2T = _layer_tc(agg_of(h1T), h1T, W2l, b2l, W2r)
    outT = _head_tc(agg_of(h2T), h2T, W3l, b3l, W3r, l1W, l1b, l2W, l2b,
                    w3p, b3p)
    return outT[:l3W.shape[0]].T


# speedup vs baseline: 3.8220x; 1.1196x over previous
import functools

import jax
import jax.numpy as jnp
from jax import lax
from jax.experimental import pallas as pl
from jax.experimental.pallas import tpu as pltpu
from jax.experimental.pallas import tpu_sc as plsc

_L = 16


def _sc_info():
    try:
        info = plsc.get_sparse_core_info()
        return info.num_cores, info.num_subcores
    except Exception:
        return 2, 16


def _segmax_t_body(nw, w, fpt, e, c, ei_hbm, ht_hbm, out_hbm,
                   hts, acc, tmp, sa, da, sb, db, sema, semb):
    nc, _ = _sc_info()
    wid = lax.axis_index("s") * nc + lax.axis_index("c")
    slab = fpt * w
    base = wid * slab
    neg = jnp.full((_L,), -jnp.inf, jnp.float32)
    lane = lax.iota(jnp.int32, _L)
    nch = e // c

    pltpu.sync_copy(ht_hbm.at[pl.ds(base, slab)], hts)

    def init_chunk(i, _):
        acc[pl.ds(i * _L, _L)] = neg
        return 0

    lax.fori_loop(0, slab // _L, init_chunk, 0)

    def start(ci, s_buf, d_buf, sem):
        pltpu.async_copy(ei_hbm.at[0, pl.ds(ci * c, c)], s_buf, sem)
        pltpu.async_copy(ei_hbm.at[1, pl.ds(ci * c, c)], d_buf, sem)

    def wait(s_buf, d_buf, sem):
        pltpu.make_async_copy(ei_hbm.at[0, pl.ds(0, c)], s_buf, sem).wait()
        pltpu.make_async_copy(ei_hbm.at[0, pl.ds(0, c)], d_buf, sem).wait()

    def process(s_buf, d_buf):
        unroll = 4

        def round_(s16, d16, hvs, rem):
            idxw = jnp.where(rem, d16, w)
            plsc.store_scatter(tmp, [idxw], lane)
            back = plsc.load_gather(tmp, [idxw])
            winner = rem & (back == lane)
            idxus = [jnp.where(winner, d16 + f * w, slab) for f in range(fpt)]
            avs = [plsc.load_gather(acc, [idxus[f]]) for f in range(fpt)]
            mxs = [jnp.maximum(avs[f], hvs[f]) for f in range(fpt)]
            for f in range(fpt):
                plsc.store_scatter(acc, [idxus[f]], mxs[f])
            return rem & (~winner)

        def group4(q, _):
            s16s = [s_buf[pl.ds((q * unroll + u) * _L, _L)]
                    for u in range(unroll)]
            d16s = [d_buf[pl.ds((q * unroll + u) * _L, _L)]
                    for u in range(unroll)]
            hvss = [[plsc.load_gather(hts, [s16s[u] + f * w])
                     for f in range(fpt)] for u in range(unroll)]
            rems = []
            for u in range(unroll):
                rems.append(round_(s16s[u], d16s[u], hvss[u],
                                   d16s[u] >= 0))

            anyrem = rems[0]
            for u in range(1, unroll):
                anyrem = anyrem | rems[u]

            @pl.when(jnp.any(anyrem))
            def _():
                for u in range(unroll):
                    gb = (q * unroll + u) * _L
                    s16 = s_buf[pl.ds(gb, _L)]
                    d16 = d_buf[pl.ds(gb, _L)]
                    hvs = [plsc.load_gather(hts, [s16 + f * w])
                           for f in range(fpt)]

                    def cond(carry):
                        rem, r = carry
                        return jnp.any(rem) & (r < _L)

                    def wbody(carry):
                        rem, r = carry
                        return round_(s16, d16, hvs, rem), r + 1

                    lax.while_loop(cond, wbody, (d16 >= 0, 0))

            return 0

        lax.fori_loop(0, c // (_L * unroll), group4, 0)

    start(0, sa, da, sema)

    def pairbody(p, _):
        start(2 * p + 1, sb, db, semb)
        wait(sa, da, sema)
        process(sa, da)
        start(2 * p + 2, sa, da, sema)
        wait(sb, db, semb)
        process(sb, db)
        return 0

    lax.fori_loop(0, (nch - 1) // 2, pairbody, 0)
    wait(sa, da, sema)
    process(sa, da)

    def fix_chunk(i, _):
        a = acc[pl.ds(i * _L, _L)]
        acc[pl.ds(i * _L, _L)] = jnp.where(a == neg, 0.0, a)
        return 0

    lax.fori_loop(0, slab // _L, fix_chunk, 0)
    pltpu.sync_copy(acc.at[pl.ds(0, slab)], out_hbm.at[pl.ds(base, slab)])


def _transpose_tc(x):
    def body(x_ref, o_ref):
        o_ref[...] = x_ref[...].T

    return pl.pallas_call(
        body,
        out_shape=jax.ShapeDtypeStruct((x.shape[1], x.shape[0]), jnp.float32),
    )(x)


def _layer_tc(aggT, hT, wl, bl, wr):
    def body(a_ref, h_ref, wl_ref, bl_ref, wr_ref, o_ref):
        o_ref[...] = (
            jnp.dot(wl_ref[...], a_ref[...], preferred_element_type=jnp.float32)
            + jnp.dot(wr_ref[...], h_ref[...], preferred_element_type=jnp.float32)
            + bl_ref[...])

    return pl.pallas_call(
        body, out_shape=jax.ShapeDtypeStruct(aggT.shape, jnp.float32),
    )(aggT, hT, wl, bl.reshape(-1, 1), wr)


def _head_tc(aggT, hT, wl, bl, wr, w1, b1, w2, b2, w3p, b3p):
    def body(a_ref, h_ref, wl_ref, bl_ref, wr_ref, w1_ref, b1_ref, w2_ref,
             b2_ref, w3_ref, b3_ref, o_ref):
        h3 = (jnp.dot(wl_ref[...], a_ref[...], preferred_element_type=jnp.float32)
              + jnp.dot(wr_ref[...], h_ref[...], preferred_element_type=jnp.float32)
              + bl_ref[...])
        t = jnp.maximum(
            jnp.dot(w1_ref[...], h3, preferred_element_type=jnp.float32)
            + b1_ref[...], 0.0)
        t = jnp.maximum(
            jnp.dot(w2_ref[...], t, preferred_element_type=jnp.float32)
            + b2_ref[...], 0.0)
        t = (jnp.dot(w3_ref[...], t, preferred_element_type=jnp.float32)
             + b3_ref[...])
        o_ref[...] = 1.0 / (1.0 + jnp.exp(-t))

    return pl.pallas_call(
        body,
        out_shape=jax.ShapeDtypeStruct((w3p.shape[0], aggT.shape[1]),
                                       jnp.float32),
    )(aggT, hT, wl, bl.reshape(-1, 1), wr, w1, b1.reshape(-1, 1), w2,
      b2.reshape(-1, 1), w3p, b3p.reshape(-1, 1))


def kernel(x, edge_index, batch, W1l, b1l, W1r, W2l, b2l, W2r, W3l, b3l, W3r,
           l1W, l1b, l2W, l2b, l3W, l3b):
    n, d = x.shape
    e = edge_index.shape[1]
    nc, ns = _sc_info()
    nw = nc * ns
    fpt = d // nw

    c = 2560
    while e % c:
        c //= 2

    mesh = plsc.VectorSubcoreMesh(core_axis_name="c", subcore_axis_name="s")

    segmax = pl.kernel(
        functools.partial(_segmax_t_body, nw, n, fpt, e, c),
        out_type=jax.ShapeDtypeStruct((d * n,), jnp.float32),
        mesh=mesh,
        scratch_types=[
            pltpu.VMEM((fpt * n,), jnp.float32),
            pltpu.VMEM((fpt * n + _L,), jnp.float32),
            pltpu.VMEM((n + _L,), jnp.int32),
            pltpu.VMEM((c,), jnp.int32),
            pltpu.VMEM((c,), jnp.int32),
            pltpu.VMEM((c,), jnp.int32),
            pltpu.VMEM((c,), jnp.int32),
            pltpu.SemaphoreType.DMA,
            pltpu.SemaphoreType.DMA,
        ],
        compiler_params=pltpu.CompilerParams(needs_layout_passes=False),
    )

    def agg_of(hT):
        return segmax(edge_index, hT.reshape(-1)).reshape(d, n)

    w3p = jnp.zeros((8, l3W.shape[1]), jnp.float32).at[:l3W.shape[0]].set(l3W)
    b3p = jnp.zeros((8,), jnp.float32).at[:l3b.shape[0]].set(l3b)

    xT = _transpose_tc(x)
    h1T = _layer_tc(agg_of(xT), xT, W1l, b1l, W1r)
    h2T = _layer_tc(agg_of(h1T), h1T, W2l, b2l, W2r)
    outT = _head_tc(agg_of(h2T), h2T, W3l, b3l, W3r, l1W, l1b, l2W, l2b,
                    w3p, b3p)
    return outT[:l3W.shape[0]].T
